# baseline (speedup 1.0000x reference)
import functools
import jax
import jax.numpy as jnp
from jax import lax
from jax.experimental import pallas as pl
from jax.experimental.pallas import tpu as pltpu
from jax.experimental.pallas import tpu_sc as plsc

NC = 2
NS = 16
NW = NC * NS
CHUNK = 512


def _mm2_body(x_ref, wa_ref, wb_ref, oa_ref, ob_ref):
    x = x_ref[...]
    oa_ref[...] = jnp.dot(x, wa_ref[...], preferred_element_type=jnp.float32)
    ob_ref[...] = jnp.dot(x, wb_ref[...], preferred_element_type=jnp.float32)


def _mid_body(xroot_ref, parts_ref, b_ref, wrel_ref, wroot_ref, hr_ref, hroot_ref):
    n = xroot_ref.shape[0]
    h = xroot_ref[...] + parts_ref[0, :n] + parts_ref[1, :n] + b_ref[...]
    h = jnp.maximum(h, 0.0)
    hr_ref[...] = jnp.dot(h, wrel_ref[...], preferred_element_type=jnp.float32)
    hroot_ref[...] = jnp.dot(h, wroot_ref[...], preferred_element_type=jnp.float32)


def _final_body(hroot_ref, parts_ref, b_ref, o_ref):
    n = hroot_ref.shape[0]
    z = hroot_ref[...] + parts_ref[0, :n] + parts_ref[1, :n] + b_ref[...]
    m = jnp.max(z, axis=1, keepdims=True)
    lse = m + jnp.log(jnp.sum(jnp.exp(z - m), axis=1, keepdims=True))
    o_ref[...] = z - lse


def _sc_scatter_body(n_ch, acc_rows, n_nodes,
                     table, srcr, dstr, zeros, out,
                     idx_s, idx_d, rowbuf0, acc, sem0):
    c = lax.axis_index("c")
    s = lax.axis_index("s")
    wid = s * NC + c
    pltpu.sync_copy(srcr.at[wid], idx_s)
    pltpu.sync_copy(dstr.at[wid], idx_d)
    zrows = acc_rows // NS
    pltpu.sync_copy(zeros.at[pl.ds(s * zrows, zrows)],
                    acc.at[pl.ds(s * zrows, zrows)])
    plsc.subcore_barrier()

    def chunk(i, carry):
        pltpu.async_copy(table.at[idx_s.at[i]], rowbuf0, sem0).wait()
        pltpu.sync_copy(rowbuf0, acc.at[idx_d.at[i]], add=True)
        return carry

    lax.fori_loop(0, n_ch, chunk, 0)
    plsc.subcore_barrier()
    pltpu.sync_copy(acc.at[pl.ds(s * zrows, zrows)],
                    out.at[c].at[pl.ds(s * zrows, zrows)])


def _make_sc_scatter(n_nodes, d, n_ch):
    acc_rows = ((n_nodes + NS * 8 - 1) // (NS * 8)) * (NS * 8) + NS * 8
    mesh = plsc.VectorSubcoreMesh(core_axis_name="c", subcore_axis_name="s")
    kern = pl.kernel(
        functools.partial(_sc_scatter_body, n_ch, acc_rows, n_nodes),
        out_type=jax.ShapeDtypeStruct((NC, acc_rows, d), jnp.float32),
        mesh=mesh,
        scratch_types=[
            pltpu.VMEM((n_ch, CHUNK), jnp.int32),
            pltpu.VMEM((n_ch, CHUNK), jnp.int32),
            pltpu.VMEM((CHUNK, d), jnp.float32),
            pltpu.VMEM_SHARED((acc_rows, d), jnp.float32),
            pltpu.SemaphoreType.DMA,
        ],
        compiler_params=pltpu.CompilerParams(use_tc_tiling_on_sc=False),
    )
    return kern, acc_rows


def kernel(x, edge_index, W1_root, W1_rel, b1, W2_root, W2_rel, b2):
    n_nodes, d_in = x.shape
    d_hid = W1_root.shape[1]
    d_out = W2_root.shape[1]
    n_edges = edge_index.shape[1]

    ept = -(-n_edges // NW)
    n_ch = -(-ept // CHUNK)
    e_pad = NW * n_ch * CHUNK

    sc1, acc_rows1 = _make_sc_scatter(n_nodes, d_hid, n_ch)
    sc2, acc_rows2 = _make_sc_scatter(n_nodes, d_out, n_ch)

    pad = e_pad - n_edges
    src = jnp.concatenate([edge_index[0], jnp.zeros((pad,), jnp.int32)])
    dst = jnp.concatenate([edge_index[1],
                           jnp.full((pad,), n_nodes, jnp.int32)])
    srcr = src.reshape(NW, n_ch, CHUNK)
    dstr = dst.reshape(NW, n_ch, CHUNK)
    zeros1 = jnp.zeros((acc_rows1, d_hid), jnp.float32)
    zeros2 = jnp.zeros((acc_rows2, d_out), jnp.float32)

    mm2 = pl.pallas_call(
        _mm2_body,
        out_shape=(jax.ShapeDtypeStruct((n_nodes, d_hid), jnp.float32),
                   jax.ShapeDtypeStruct((n_nodes, d_hid), jnp.float32)),
    )
    xr1, xroot = mm2(x, W1_rel, W1_root)

    parts1 = sc1(xr1, srcr, dstr, zeros1)

    mid = pl.pallas_call(
        _mid_body,
        out_shape=(jax.ShapeDtypeStruct((n_nodes, d_out), jnp.float32),
                   jax.ShapeDtypeStruct((n_nodes, d_out), jnp.float32)),
    )
    hr, hroot = mid(xroot, parts1, b1.reshape(1, d_hid), W2_rel, W2_root)

    parts2 = sc2(hr, srcr, dstr, zeros2)

    final = pl.pallas_call(
        _final_body,
        out_shape=jax.ShapeDtypeStruct((n_nodes, d_out), jnp.float32),
    )
    return final(hroot, parts2, b2.reshape(1, d_out))


# speedup vs baseline: 1.8341x; 1.8341x over previous
import functools
import jax
import jax.numpy as jnp
from jax import lax
from jax.experimental import pallas as pl
from jax.experimental.pallas import tpu as pltpu
from jax.experimental.pallas import tpu_sc as plsc

NC = 2
NS = 16
NW = NC * NS
CHUNK = 128


def _mm2_body(x_ref, wa_ref, wb_ref, oa_ref, ob_ref):
    x = x_ref[...]
    n = x_ref.shape[0]
    oa_ref[:n] = jnp.dot(x, wa_ref[...], preferred_element_type=jnp.float32)
    if oa_ref.shape[0] > n:
        oa_ref[n:] = jnp.zeros((oa_ref.shape[0] - n, oa_ref.shape[1]),
                               jnp.float32)
    ob_ref[...] = jnp.dot(x, wb_ref[...], preferred_element_type=jnp.float32)


def _mid_body(xroot_ref, parts_ref, b_ref, wrel_ref, wroot_ref, hr_ref, hroot_ref):
    n = xroot_ref.shape[0]
    h = xroot_ref[...] + parts_ref[0, :n] + parts_ref[1, :n] + b_ref[...]
    h = jnp.maximum(h, 0.0)
    hr_ref[:n] = jnp.dot(h, wrel_ref[...], preferred_element_type=jnp.float32)
    if hr_ref.shape[0] > n:
        hr_ref[n:] = jnp.zeros((hr_ref.shape[0] - n, hr_ref.shape[1]),
                               jnp.float32)
    hroot_ref[...] = jnp.dot(h, wroot_ref[...], preferred_element_type=jnp.float32)


def _final_body(hroot_ref, parts_ref, b_ref, o_ref):
    n = hroot_ref.shape[0]
    z = hroot_ref[...] + parts_ref[0, :n] + parts_ref[1, :n] + b_ref[...]
    m = jnp.max(z, axis=1, keepdims=True)
    lse = m + jnp.log(jnp.sum(jnp.exp(z - m), axis=1, keepdims=True))
    o_ref[...] = z - lse


def _sc_scatter_body(n_ch, acc_rows, n_nodes,
                     table, srcr, dstr, zeros, out,
                     idx_s, idx_d, rowbuf0, tab_sh, acc, sem0):
    c = lax.axis_index("c")
    s = lax.axis_index("s")
    wid = s * NC + c
    pltpu.sync_copy(srcr.at[wid], idx_s)
    pltpu.sync_copy(dstr.at[wid], idx_d)
    trows = table.shape[0] // NS
    pltpu.sync_copy(table.at[pl.ds(s * trows, trows)],
                    tab_sh.at[pl.ds(s * trows, trows)])
    zrows = acc_rows // NS
    pltpu.sync_copy(zeros.at[pl.ds(s * zrows, zrows)],
                    acc.at[pl.ds(s * zrows, zrows)])
    plsc.subcore_barrier()

    def chunk(i, carry):
        pltpu.async_copy(tab_sh.at[idx_s.at[i]], rowbuf0, sem0).wait()
        pltpu.sync_copy(rowbuf0, acc.at[idx_d.at[i]], add=True)
        return carry

    lax.fori_loop(0, n_ch, chunk, 0)
    plsc.subcore_barrier()
    pltpu.sync_copy(acc.at[pl.ds(s * zrows, zrows)],
                    out.at[c].at[pl.ds(s * zrows, zrows)])


def _make_sc_scatter(n_nodes, tpad, d, n_ch):
    acc_rows = ((n_nodes + NS * 8 - 1) // (NS * 8)) * (NS * 8) + NS * 8
    mesh = plsc.VectorSubcoreMesh(core_axis_name="c", subcore_axis_name="s")
    kern = pl.kernel(
        functools.partial(_sc_scatter_body, n_ch, acc_rows, n_nodes),
        out_type=jax.ShapeDtypeStruct((NC, acc_rows, d), jnp.float32),
        mesh=mesh,
        scratch_types=[
            pltpu.VMEM((n_ch, CHUNK), jnp.int32),
            pltpu.VMEM((n_ch, CHUNK), jnp.int32),
            pltpu.VMEM((CHUNK, d), jnp.float32),
            pltpu.VMEM_SHARED((tpad, d), jnp.float32),
            pltpu.VMEM_SHARED((acc_rows, d), jnp.float32),
            pltpu.SemaphoreType.DMA,
        ],
        compiler_params=pltpu.CompilerParams(use_tc_tiling_on_sc=False),
    )
    return kern, acc_rows


def kernel(x, edge_index, W1_root, W1_rel, b1, W2_root, W2_rel, b2):
    n_nodes, d_in = x.shape
    d_hid = W1_root.shape[1]
    d_out = W2_root.shape[1]
    n_edges = edge_index.shape[1]

    ept = -(-n_edges // NW)
    n_ch = -(-ept // CHUNK)
    e_pad = NW * n_ch * CHUNK

    tpad = ((n_nodes + NS - 1) // NS) * NS
    sc1, acc_rows1 = _make_sc_scatter(n_nodes, tpad, d_hid, n_ch)
    sc2, acc_rows2 = _make_sc_scatter(n_nodes, tpad, d_out, n_ch)

    pad = e_pad - n_edges
    src = jnp.concatenate([edge_index[0], jnp.zeros((pad,), jnp.int32)])
    dst = jnp.concatenate([edge_index[1],
                           jnp.full((pad,), n_nodes, jnp.int32)])
    srcr = src.reshape(NW, n_ch, CHUNK)
    dstr = dst.reshape(NW, n_ch, CHUNK)
    zeros1 = jnp.zeros((acc_rows1, d_hid), jnp.float32)
    zeros2 = jnp.zeros((acc_rows2, d_out), jnp.float32)

    mm2 = pl.pallas_call(
        _mm2_body,
        out_shape=(jax.ShapeDtypeStruct((tpad, d_hid), jnp.float32),
                   jax.ShapeDtypeStruct((n_nodes, d_hid), jnp.float32)),
    )
    xr1, xroot = mm2(x, W1_rel, W1_root)

    parts1 = sc1(xr1, srcr, dstr, zeros1)

    mid = pl.pallas_call(
        _mid_body,
        out_shape=(jax.ShapeDtypeStruct((tpad, d_out), jnp.float32),
                   jax.ShapeDtypeStruct((n_nodes, d_out), jnp.float32)),
    )
    hr, hroot = mid(xroot, parts1, b1.reshape(1, d_hid), W2_rel, W2_root)

    parts2 = sc2(hr, srcr, dstr, zeros2)

    final = pl.pallas_call(
        _final_body,
        out_shape=jax.ShapeDtypeStruct((n_nodes, d_out), jnp.float32),
    )
    return final(hroot, parts2, b2.reshape(1, d_out))


# speedup vs baseline: 1.9342x; 1.0546x over previous
import functools
import jax
import jax.numpy as jnp
from jax import lax
from jax.experimental import pallas as pl
from jax.experimental.pallas import tpu as pltpu
from jax.experimental.pallas import tpu_sc as plsc

NC = 2
NS = 16
NW = NC * NS
CHUNK = 128


def _mm2_body(x_ref, wa_ref, wb_ref, oa_ref, ob_ref):
    x = x_ref[...]
    n = x_ref.shape[0]
    oa_ref[:n] = jnp.dot(x, wa_ref[...], preferred_element_type=jnp.float32)
    if oa_ref.shape[0] > n:
        oa_ref[n:] = jnp.zeros((oa_ref.shape[0] - n, oa_ref.shape[1]),
                               jnp.float32)
    ob_ref[...] = jnp.dot(x, wb_ref[...], preferred_element_type=jnp.float32)


def _mid_body(xroot_ref, parts_ref, b_ref, wrel_ref, wroot_ref, hr_ref, hroot_ref):
    n = xroot_ref.shape[0]
    h = xroot_ref[...] + parts_ref[0, :n] + parts_ref[1, :n] + b_ref[...]
    h = jnp.maximum(h, 0.0)
    hr_ref[:n] = jnp.dot(h, wrel_ref[...], preferred_element_type=jnp.float32)
    if hr_ref.shape[0] > n:
        hr_ref[n:] = jnp.zeros((hr_ref.shape[0] - n, hr_ref.shape[1]),
                               jnp.float32)
    hroot_ref[...] = jnp.dot(h, wroot_ref[...], preferred_element_type=jnp.float32)


def _final_body(hroot_ref, parts_ref, b_ref, o_ref):
    n = hroot_ref.shape[0]
    z = hroot_ref[...] + parts_ref[0, :n] + parts_ref[1, :n] + b_ref[...]
    m = jnp.max(z, axis=1, keepdims=True)
    lse = m + jnp.log(jnp.sum(jnp.exp(z - m), axis=1, keepdims=True))
    o_ref[...] = z - lse


def _sc_scatter_body(ept, acc_rows, n_nodes, d,
                     table, edges, out, idx_s, idx_d, rowbuf0, tab_sh, acc,
                     sem0):
    c = lax.axis_index("c")
    s = lax.axis_index("s")
    wid = s * NC + c
    base = wid * ept
    pltpu.sync_copy(edges.at[0].at[pl.ds(base, ept)], idx_s)
    pltpu.sync_copy(edges.at[1].at[pl.ds(base, ept)], idx_d)
    trows = table.shape[0] // NS
    pltpu.sync_copy(table.at[pl.ds(s * trows, trows)],
                    tab_sh.at[pl.ds(s * trows, trows)])
    zvec = jnp.zeros((16,), jnp.float32)

    def zfill(i, carry):
        rowbuf0[i // (d // 16), pl.ds((i % (d // 16)) * 16, 16)] = zvec
        return carry

    lax.fori_loop(0, CHUNK * d // 16, zfill, 0)
    zrows = acc_rows // NS
    for k in range(zrows // CHUNK):
        pltpu.sync_copy(rowbuf0, acc.at[pl.ds(s * zrows + k * CHUNK, CHUNK)])
    plsc.subcore_barrier()

    n_full = ept // CHUNK
    tail = ept - n_full * CHUNK

    def chunk(i, carry):
        ii = i * CHUNK
        pltpu.async_copy(tab_sh.at[idx_s.at[pl.ds(ii, CHUNK)]], rowbuf0,
                         sem0).wait()
        pltpu.sync_copy(rowbuf0, acc.at[idx_d.at[pl.ds(ii, CHUNK)]], add=True)
        return carry

    lax.fori_loop(0, n_full, chunk, 0)
    if tail:
        tt = n_full * CHUNK
        pltpu.async_copy(tab_sh.at[idx_s.at[pl.ds(tt, tail)]],
                         rowbuf0.at[pl.ds(0, tail)], sem0).wait()
        pltpu.sync_copy(rowbuf0.at[pl.ds(0, tail)],
                        acc.at[idx_d.at[pl.ds(tt, tail)]], add=True)
    plsc.subcore_barrier()
    pltpu.sync_copy(acc.at[pl.ds(s * zrows, zrows)],
                    out.at[c].at[pl.ds(s * zrows, zrows)])


def _make_sc_scatter(n_nodes, tpad, d, ept):
    acc_rows = ((n_nodes + NS * CHUNK - 1) // (NS * CHUNK)) * (NS * CHUNK)
    mesh = plsc.VectorSubcoreMesh(core_axis_name="c", subcore_axis_name="s")
    kern = pl.kernel(
        functools.partial(_sc_scatter_body, ept, acc_rows, n_nodes, d),
        out_type=jax.ShapeDtypeStruct((NC, acc_rows, d), jnp.float32),
        mesh=mesh,
        scratch_types=[
            pltpu.VMEM((ept,), jnp.int32),
            pltpu.VMEM((ept,), jnp.int32),
            pltpu.VMEM((CHUNK, d), jnp.float32),
            pltpu.VMEM_SHARED((tpad, d), jnp.float32),
            pltpu.VMEM_SHARED((acc_rows, d), jnp.float32),
            pltpu.SemaphoreType.DMA,
        ],
        compiler_params=pltpu.CompilerParams(use_tc_tiling_on_sc=False),
    )
    return kern, acc_rows


def kernel(x, edge_index, W1_root, W1_rel, b1, W2_root, W2_rel, b2):
    n_nodes, d_in = x.shape
    d_hid = W1_root.shape[1]
    d_out = W2_root.shape[1]
    n_edges = edge_index.shape[1]

    ept = n_edges // NW

    tpad = ((n_nodes + NS - 1) // NS) * NS
    sc1, acc_rows1 = _make_sc_scatter(n_nodes, tpad, d_hid, ept)
    sc2, acc_rows2 = _make_sc_scatter(n_nodes, tpad, d_out, ept)

    mm2 = pl.pallas_call(
        _mm2_body,
        out_shape=(jax.ShapeDtypeStruct((tpad, d_hid), jnp.float32),
                   jax.ShapeDtypeStruct((n_nodes, d_hid), jnp.float32)),
    )
    xr1, xroot = mm2(x, W1_rel, W1_root)

    parts1 = sc1(xr1, edge_index)

    mid = pl.pallas_call(
        _mid_body,
        out_shape=(jax.ShapeDtypeStruct((tpad, d_out), jnp.float32),
                   jax.ShapeDtypeStruct((n_nodes, d_out), jnp.float32)),
    )
    hr, hroot = mid(xroot, parts1, b1.reshape(1, d_hid), W2_rel, W2_root)

    parts2 = sc2(hr, edge_index)

    final = pl.pallas_call(
        _final_body,
        out_shape=jax.ShapeDtypeStruct((n_nodes, d_out), jnp.float32),
    )
    return final(hroot, parts2, b2.reshape(1, d_out))
